# baseline (speedup 1.0000x reference)
import functools

import jax
import jax.numpy as jnp
from jax.experimental import pallas as pl

_NE = 64
_TK = 32
_CF = 1.25
_EPS = 1e-06


def _gate_kernel(x_ref, wt_ref, b_ref, o_ref, *, capacity):
    B, S, D = x_ref.shape
    E = _NE
    x = x_ref[...].reshape(B * S, D)
    logits = jnp.dot(x, wt_ref[...], preferred_element_type=jnp.float32)
    logits = logits + b_ref[...]
    m = jnp.max(logits, axis=-1, keepdims=True)
    ex = jnp.exp(logits - m)
    g = ex / jnp.sum(ex, axis=-1, keepdims=True)
    eidx = jax.lax.broadcasted_iota(jnp.int32, (1, E), 1)
    rank = jnp.zeros(g.shape, jnp.int32)
    for j in range(E):
        gj = g[:, j:j + 1]
        beats = (gj > g) | ((gj == g) & (j < eidx))
        rank = rank + beats.astype(jnp.int32)
    masked = jnp.where(rank < _TK, g, 0.0).reshape(B, S, E)
    denom = jnp.sum(masked, axis=0, keepdims=True) + _EPS
    o_ref[...] = masked / denom * capacity


def kernel(X, W, b):
    B, S, D = X.shape
    capacity = int(_CF * B)
    Wt = W.T
    b2 = b.reshape(1, _NE)
    SB = 256
    grid = (S // SB,)
    return pl.pallas_call(
        functools.partial(_gate_kernel, capacity=float(capacity)),
        grid=grid,
        in_specs=[
            pl.BlockSpec((B, SB, D), lambda i: (0, i, 0)),
            pl.BlockSpec((D, _NE), lambda i: (0, 0)),
            pl.BlockSpec((1, _NE), lambda i: (0, 0)),
        ],
        out_specs=pl.BlockSpec((B, SB, _NE), lambda i: (0, i, 0)),
        out_shape=jax.ShapeDtypeStruct((B, S, _NE), jnp.float32),
    )(X, Wt, b2)
